# baseline (speedup 1.0000x reference)
import functools

import jax
import jax.numpy as jnp
from jax import lax
from jax.experimental import pallas as pl
from jax.experimental.pallas import tpu as pltpu
from jax.experimental.pallas import tpu_sc as plsc

C = 4096
NUM_PEDESTALS = 16
PED_FEATS = 4
SP_FEATS = 2
OUT_FEATS = PED_FEATS + SP_FEATS

_info = plsc.get_sparse_core_info()
NC, NS, L = _info.num_cores, _info.num_subcores, _info.num_lanes
NW = NC * NS
CPW = C // NW
FLAT = CPW * OUT_FEATS
TBL = NUM_PEDESTALS * PED_FEATS
SP_OFF = TBL
STEPS = FLAT // L


def _sc_body(idx_hbm, sp_hbm, tbl_hbm, out_hbm, idx_v, comb_v, out_v):
    wid = lax.axis_index("s") * NC + lax.axis_index("c")
    base = wid * CPW

    pltpu.sync_copy(idx_hbm.at[pl.ds(base, CPW)], idx_v)
    pltpu.sync_copy(tbl_hbm, comb_v.at[pl.ds(0, TBL)])
    pltpu.sync_copy(sp_hbm.at[pl.ds(base * SP_FEATS, CPW * SP_FEATS)],
                    comb_v.at[pl.ds(SP_OFF, CPW * SP_FEATS)])

    lanes = lax.iota(jnp.int32, L)
    for t in range(STEPS):
        j = lanes + t * L
        row = j // OUT_FEATS
        col = j - row * OUT_FEATS
        ped = plsc.load_gather(idx_v, [row])
        src = jnp.where(col < PED_FEATS,
                        ped * PED_FEATS + col,
                        SP_OFF + row * SP_FEATS + (col - PED_FEATS))
        out_v[pl.ds(t * L, L)] = plsc.load_gather(comb_v, [src])

    pltpu.sync_copy(out_v, out_hbm.at[pl.ds(base * OUT_FEATS, FLAT)])


_sc_call = functools.partial(
    pl.kernel,
    mesh=plsc.VectorSubcoreMesh(core_axis_name="c", subcore_axis_name="s"),
    out_type=jax.ShapeDtypeStruct((C * OUT_FEATS,), jnp.float32),
    scratch_types=[
        pltpu.VMEM((CPW,), jnp.int32),
        pltpu.VMEM((TBL + CPW * SP_FEATS,), jnp.float32),
        pltpu.VMEM((FLAT,), jnp.float32),
    ],
    compiler_params=pltpu.CompilerParams(needs_layout_passes=False),
)(_sc_body)


@jax.jit
def kernel(pedestals, spatial_embeddings, pedestal_table):
    idx = pedestals.astype(jnp.int32)
    sp = spatial_embeddings.reshape(-1)
    tbl = pedestal_table.reshape(-1)
    out = _sc_call(idx, sp, tbl)
    return out.reshape(C, OUT_FEATS)


# speedup vs baseline: 1.0384x; 1.0384x over previous
import functools

import jax
import jax.numpy as jnp
from jax import lax
from jax.experimental import pallas as pl
from jax.experimental.pallas import tpu as pltpu
from jax.experimental.pallas import tpu_sc as plsc

C = 4096
NUM_PEDESTALS = 16
PED_FEATS = 4
SP_FEATS = 2
OUT_FEATS = PED_FEATS + SP_FEATS

_info = plsc.get_sparse_core_info()
NC, NS, L = _info.num_cores, _info.num_subcores, _info.num_lanes
NW = NC * NS
CPW = C // NW
FLAT = CPW * OUT_FEATS
TBL = NUM_PEDESTALS * PED_FEATS
SP_OFF = TBL
STEPS = FLAT // L


def _sc_body(idx_hbm, sp_hbm, tbl_hbm, out_hbm, idx_v, comb_v, out_v, sem):
    wid = lax.axis_index("s") * NC + lax.axis_index("c")
    base = wid * CPW

    cp_idx = pltpu.async_copy(idx_hbm.at[pl.ds(base, CPW)], idx_v, sem)
    cp_tbl = pltpu.async_copy(tbl_hbm, comb_v.at[pl.ds(0, TBL)], sem)
    cp_sp = pltpu.async_copy(
        sp_hbm.at[pl.ds(base * SP_FEATS, CPW * SP_FEATS)],
        comb_v.at[pl.ds(SP_OFF, CPW * SP_FEATS)], sem)
    cp_idx.wait()
    cp_tbl.wait()
    cp_sp.wait()

    lanes = lax.iota(jnp.int32, L)
    for t in range(STEPS):
        j = lanes + t * L
        row = j // OUT_FEATS
        col = j - row * OUT_FEATS
        ped = plsc.load_gather(idx_v, [row])
        src = jnp.where(col < PED_FEATS,
                        ped * PED_FEATS + col,
                        SP_OFF + row * SP_FEATS + (col - PED_FEATS))
        out_v[pl.ds(t * L, L)] = plsc.load_gather(comb_v, [src])

    pltpu.sync_copy(out_v, out_hbm.at[pl.ds(base * OUT_FEATS, FLAT)])


_sc_call = functools.partial(
    pl.kernel,
    mesh=plsc.VectorSubcoreMesh(core_axis_name="c", subcore_axis_name="s"),
    out_type=jax.ShapeDtypeStruct((C * OUT_FEATS,), jnp.float32),
    scratch_types=[
        pltpu.VMEM((CPW,), jnp.int32),
        pltpu.VMEM((TBL + CPW * SP_FEATS,), jnp.float32),
        pltpu.VMEM((FLAT,), jnp.float32),
        pltpu.SemaphoreType.DMA,
    ],
    compiler_params=pltpu.CompilerParams(
        needs_layout_passes=False,
        disable_bounds_checks=True,
        skip_device_barrier=True,
    ),
)(_sc_body)


@jax.jit
def kernel(pedestals, spatial_embeddings, pedestal_table):
    idx = pedestals.astype(jnp.int32)
    sp = spatial_embeddings.reshape(-1)
    tbl = pedestal_table.reshape(-1)
    out = _sc_call(idx, sp, tbl)
    return out.reshape(C, OUT_FEATS)
